# baseline (speedup 1.0000x reference)
import functools

import jax
import jax.numpy as jnp
from jax import lax
from jax.experimental import pallas as pl
from jax.experimental.pallas import tpu as pltpu
from jax.experimental.pallas import tpu_sc as plsc

N = 2048
D = 768
E = 6
K = 2
F = 3072
TM = 256
MAXT = (N * K) // TM + (E - 1)
MAXR = MAXT * TM
TN = 256


def _router_body(x_ref, wr_ref, i0_ref, i1_ref, p0_ref, p1_ref, aux_ref,
                 xc_ref, acc_ref):
    step = pl.program_id(0)
    xc_ref[...] = x_ref[...]

    @pl.when(step == 0)
    def _():
        acc_ref[...] = jnp.zeros_like(acc_ref)

    x = x_ref[...]
    wr = wr_ref[...]
    logits = lax.dot_general(x, wr, (((1,), (1,)), ((), ())),
                             preferred_element_type=jnp.float32)
    m = jnp.max(logits, axis=1, keepdims=True)
    ex = jnp.exp(logits - m)
    s = jnp.sum(ex, axis=1, keepdims=True)
    probs = ex / s

    iota = lax.broadcasted_iota(jnp.int32, (TN, E), 1)
    m1 = jnp.max(probs, axis=1, keepdims=True)
    i1 = jnp.min(jnp.where(probs == m1, iota, E), axis=1, keepdims=True)
    probs2 = jnp.where(iota == i1, -1.0, probs)
    m2 = jnp.max(probs2, axis=1, keepdims=True)
    i2 = jnp.min(jnp.where(probs2 == m2, iota, E), axis=1, keepdims=True)
    denom = m1 + m2
    i0_ref[...] = i1
    i1_ref[...] = i2
    p0_ref[...] = m1 / denom
    p1_ref[...] = m2 / denom

    acc_ref[0:1, 0:E] += jnp.sum(probs, axis=0, keepdims=True)

    @pl.when(step == pl.num_programs(0) - 1)
    def _():
        colmean = acc_ref[0:1, 0:E] / float(N)
        d = colmean - (1.0 / E)
        aux_ref[...] = (0.01 * jnp.sum(d * d) / float(E)).reshape(1, 1)


def _router(x_flat, Wr):
    grid = N // TN
    out_shapes = (
        jax.ShapeDtypeStruct((N, 1), jnp.int32),
        jax.ShapeDtypeStruct((N, 1), jnp.int32),
        jax.ShapeDtypeStruct((N, 1), jnp.float32),
        jax.ShapeDtypeStruct((N, 1), jnp.float32),
        jax.ShapeDtypeStruct((1, 1), jnp.float32),
        jax.ShapeDtypeStruct((N, D), jnp.float32),
    )
    tok_spec = pl.BlockSpec((TN, 1), lambda i: (i, 0))
    i0, i1, p0, p1, aux, x_copy = pl.pallas_call(
        _router_body,
        grid=(grid,),
        in_specs=[
            pl.BlockSpec((TN, D), lambda i: (i, 0)),
            pl.BlockSpec((E, D), lambda i: (0, 0)),
        ],
        out_specs=(tok_spec, tok_spec, tok_spec, tok_spec,
                   pl.BlockSpec((1, 1), lambda i: (0, 0)),
                   pl.BlockSpec((TN, D), lambda i: (i, 0))),
        out_shape=out_shapes,
        scratch_shapes=[pltpu.VMEM((8, 128), jnp.float32)],
    )(x_flat, Wr)
    return (i0[:, 0], i1[:, 0], p0[:, 0], p1[:, 0], aux[0, 0], x_copy)


L = 16
NV = N // L
MAXTP = 32


def _dispatch(i0, i1, p0, p1):
    mesh = plsc.VectorSubcoreMesh(core_axis_name="c", subcore_axis_name="s")
    SEGCAP = N + TM
    ZCH = MAXR // L

    @functools.partial(
        pl.kernel,
        out_type=(
            jax.ShapeDtypeStruct((MAXR,), jnp.int32),
            jax.ShapeDtypeStruct((MAXR,), jnp.float32),
            jax.ShapeDtypeStruct((MAXTP,), jnp.int32),
            jax.ShapeDtypeStruct((MAXTP,), jnp.int32),
            jax.ShapeDtypeStruct((N,), jnp.int32),
            jax.ShapeDtypeStruct((N,), jnp.int32),
        ),
        mesh=mesh,
        scratch_types=[
            pltpu.VMEM((N,), jnp.int32),
            pltpu.VMEM((N,), jnp.int32),
            pltpu.VMEM((N,), jnp.float32),
            pltpu.VMEM((N,), jnp.float32),
            pltpu.VMEM((SEGCAP,), jnp.int32),
            pltpu.VMEM((SEGCAP,), jnp.float32),
            pltpu.VMEM((N,), jnp.int32),
            pltpu.VMEM((N,), jnp.int32),
            pltpu.VMEM((MAXTP,), jnp.int32),
            pltpu.VMEM((MAXTP,), jnp.int32),
            pltpu.VMEM((N,), jnp.int32),
            pltpu.VMEM_SHARED((N,), jnp.int32),
            pltpu.VMEM_SHARED((N,), jnp.int32),
        ],
        compiler_params=pltpu.CompilerParams(needs_layout_passes=False),
    )
    def disp(i0_h, i1_h, p0_h, p1_h, g_h, ps_h, te_h, tv_h, pos0_h, pos1_h,
             vi0, vi1, vp0, vp1, seg_tok, seg_prob, vpos0, vpos1, texp_v,
             tval_v, idx_iota, pos0_sh, pos1_sh):
        cid = lax.axis_index("c")
        sid = lax.axis_index("s")
        z16i = jnp.zeros((L,), jnp.int32)
        z16f = jnp.zeros((L,), jnp.float32)

        @pl.when(cid == 0)
        def _():
            @pl.when(sid < E)
            def _():
                def bz(j, c):
                    vpos0[pl.ds(j * L, L)] = z16i
                    vpos1[pl.ds(j * L, L)] = z16i
                    idx_iota[pl.ds(j * L, L)] = lax.iota(jnp.int32, L) + j * L
                    return c

                lax.fori_loop(0, NV, bz, 0)

            def bseg(j, c):
                seg_tok[pl.ds(j * L, L)] = z16i
                seg_prob[pl.ds(j * L, L)] = z16f
                return c

            lax.fori_loop(0, SEGCAP // L, bseg, 0)

            pltpu.sync_copy(seg_tok.at[pl.ds(0, ZCH)],
                            g_h.at[pl.ds(sid * ZCH, ZCH)])
            pltpu.sync_copy(seg_prob.at[pl.ds(0, ZCH)],
                            ps_h.at[pl.ds(sid * ZCH, ZCH)])

            @pl.when(sid == 0)
            def _():
                pltpu.sync_copy(vpos0, pos0_sh)
                pltpu.sync_copy(vpos1, pos1_sh)

            @pl.when(sid < E)
            def _():
                pltpu.sync_copy(i0_h, vi0)
                pltpu.sync_copy(i1_h, vi1)
                pltpu.sync_copy(p0_h, vp0)
                pltpu.sync_copy(p1_h, vp1)

            plsc.subcore_barrier()

            @pl.when(sid < E)
            def _():
                def body1(j, accs):
                    off = j * L
                    v0 = vi0[pl.ds(off, L)]
                    v1 = vi1[pl.ds(off, L)]
                    return tuple(
                        accs[e]
                        + jnp.where(v0 == e, 1, 0).astype(jnp.int32)
                        + jnp.where(v1 == e, 1, 0).astype(jnp.int32)
                        for e in range(E))

                accs = lax.fori_loop(0, NV, body1,
                                     tuple(z16i for _ in range(E)))
                cnts = [jnp.sum(accs[e]) for e in range(E)]

                p_run = jnp.int32(0)
                t_run = jnp.int32(0)
                ppad, toff, tiles_l = [], [], []
                for e in range(E):
                    t_e = (cnts[e] + (TM - 1)) // TM
                    ppad.append(p_run)
                    toff.append(t_run)
                    tiles_l.append(t_e)
                    p_run = p_run + t_e * TM
                    t_run = t_run + t_e

                my_base = jnp.int32(0)
                my_tiles = jnp.int32(0)
                for e in range(E):
                    my_base = jnp.where(sid == e, ppad[e], my_base)
                    my_tiles = jnp.where(sid == e, tiles_l[e], my_tiles)

                def proc(v, pvals, n_ids, lbase, vpos):
                    m = v == sid
                    ones = jnp.where(m, 1, 0).astype(jnp.int32)
                    pref = plsc.cumsum(ones)
                    lpos = lbase + pref - 1
                    plsc.store_scatter(seg_tok, [lpos], n_ids, mask=m)
                    plsc.store_scatter(seg_prob, [lpos], pvals, mask=m)
                    plsc.store_scatter(vpos, [n_ids], lpos + my_base, mask=m)
                    return lbase + plsc.all_reduce_population_count(m)

                def body2(j, lbase):
                    off = j * L
                    n_ids = lax.iota(jnp.int32, L) + off
                    lbase = proc(vi0[pl.ds(off, L)], vp0[pl.ds(off, L)],
                                 n_ids, lbase, vpos0)
                    lbase = proc(vi1[pl.ds(off, L)], vp1[pl.ds(off, L)],
                                 n_ids, lbase, vpos1)
                    return lbase

                lax.fori_loop(0, NV, body2, z16i)

                def bcopy(j, c):
                    pltpu.sync_copy(
                        seg_tok.at[pl.ds(j * TM, TM)],
                        g_h.at[pl.ds(my_base + j * TM, TM)])
                    pltpu.sync_copy(
                        seg_prob.at[pl.ds(j * TM, TM)],
                        ps_h.at[pl.ds(my_base + j * TM, TM)])
                    return c

                lax.fori_loop(0, my_tiles, bcopy, 0)

                pltpu.sync_copy(vpos0, pos0_sh.at[idx_iota], add=True)
                pltpu.sync_copy(vpos1, pos1_sh.at[idx_iota], add=True)

                @pl.when(sid == 0)
                def _():
                    for half in range(MAXTP // L):
                        t16 = lax.iota(jnp.int32, L) + half * L
                        texp = jnp.full((L,), -1, jnp.int32)
                        for e in range(E):
                            texp = texp + jnp.where(
                                t16 >= toff[e], 1, 0).astype(jnp.int32)
                        tval = jnp.where(t16 < t_run, 1, 0).astype(jnp.int32)
                        texp_v[pl.ds(half * L, L)] = texp
                        tval_v[pl.ds(half * L, L)] = tval
                    pltpu.sync_copy(texp_v, te_h)
                    pltpu.sync_copy(tval_v, tv_h)

            plsc.subcore_barrier()

            @pl.when(sid == 0)
            def _():
                pltpu.sync_copy(pos0_sh, pos0_h)
                pltpu.sync_copy(pos1_sh, pos1_h)

    g, ps, te, tv, pos0, pos1 = disp(i0, i1, p0, p1)
    return g, ps, te[:MAXT], tv[:MAXT], pos0, pos1


def _sc_gather_rows(x_flat, gather_idx):
    mesh = plsc.VectorSubcoreMesh(core_axis_name="c", subcore_axis_name="s")
    rpw = MAXR // 32
    ch = 56
    nch = rpw // ch

    @functools.partial(
        pl.kernel,
        out_type=jax.ShapeDtypeStruct((MAXR, D), jnp.float32),
        mesh=mesh,
        scratch_types=[
            pltpu.VMEM((ch,), jnp.int32),
            pltpu.VMEM((ch,), jnp.int32),
            pltpu.VMEM((ch,), jnp.int32),
            pltpu.VMEM((ch, D), jnp.float32),
            pltpu.VMEM((ch, D), jnp.float32),
            pltpu.SemaphoreType.DMA,
            pltpu.SemaphoreType.DMA,
            pltpu.SemaphoreType.DMA,
            pltpu.SemaphoreType.DMA,
        ],
    )
    def gat(x_h, gi_h, out_h, idx0, idx1, idx2, rows0, rows1, g0, g1, w0,
            w1):
        wid = lax.axis_index("s") * 2 + lax.axis_index("c")
        base = wid * rpw
        idxs = (idx0, idx1, idx2)
        for c in range(nch):
            pltpu.sync_copy(gi_h.at[pl.ds(base + c * ch, ch)], idxs[c])
        bufs = (rows0, rows1)
        gsems = (g0, g1)
        wsems = (w0, w1)
        gathers = [None] * nch
        writes = [None] * nch
        for c in range(nch):
            if c >= 2 and writes[c - 2] is not None:
                writes[c - 2].wait()
            gathers[c] = pltpu.async_copy(
                x_h.at[idxs[c]], bufs[c % 2], gsems[c % 2])
            if c >= 1:
                gathers[c - 1].wait()
                writes[c - 1] = pltpu.async_copy(
                    bufs[(c - 1) % 2],
                    out_h.at[pl.ds(base + (c - 1) * ch, ch)],
                    wsems[(c - 1) % 2])
        gathers[nch - 1].wait()
        writes[nch - 1] = pltpu.async_copy(
            bufs[(nch - 1) % 2],
            out_h.at[pl.ds(base + (nch - 1) * ch, ch)], wsems[(nch - 1) % 2])
        for c in (nch - 2, nch - 1):
            writes[c].wait()

    return gat(x_flat, gather_idx)


def _sc_combine(rows, pos0, pos1):
    mesh = plsc.VectorSubcoreMesh(core_axis_name="c", subcore_axis_name="s")
    tpw = N // 32

    @functools.partial(
        pl.kernel,
        out_type=jax.ShapeDtypeStruct((N, D), jnp.float32),
        mesh=mesh,
        scratch_types=[
            pltpu.VMEM((tpw,), jnp.int32),
            pltpu.VMEM((tpw,), jnp.int32),
            pltpu.VMEM((tpw, D), jnp.float32),
            pltpu.VMEM((tpw, D), jnp.float32),
            pltpu.SemaphoreType.DMA,
            pltpu.SemaphoreType.DMA,
        ],
    )
    def comb(rows_h, pos0_h, pos1_h, out_h, idx0_v, idx1_v, a_v, b_v,
             sem0, sem1):
        wid = lax.axis_index("s") * 2 + lax.axis_index("c")
        base = wid * tpw
        pltpu.sync_copy(pos0_h.at[pl.ds(base, tpw)], idx0_v)
        pltpu.sync_copy(pos1_h.at[pl.ds(base, tpw)], idx1_v)
        cp0 = pltpu.async_copy(rows_h.at[idx0_v], a_v, sem0)
        cp1 = pltpu.async_copy(rows_h.at[idx1_v], b_v, sem1)
        cp0.wait()
        cp1.wait()

        @plsc.parallel_loop(0, tpw, 1, unroll=2)
        def _add(r):
            for c in range(D // L):
                sl = pl.ds(c * L, L)
                a_v[r, sl] = a_v[r, sl] + b_v[r, sl]

        pltpu.sync_copy(a_v, out_h.at[pl.ds(base, tpw)])

    return comb(rows, pos0, pos1)


def _gemm_body(te_ref, tv_ref, x_ref, w1_ref, w2_ref, pr_ref, o_ref):
    t = pl.program_id(0)

    @pl.when(tv_ref[t] > 0)
    def _():
        x = x_ref[...]
        w1 = w1_ref[0]
        h = lax.dot_general(x, w1, (((1,), (1,)), ((), ())),
                            preferred_element_type=jnp.float32)
        g = 0.5 * h * (1.0 + lax.erf(h * 0.7071067811865476))
        w2 = w2_ref[0]
        y = lax.dot_general(g, w2, (((1,), (1,)), ((), ())),
                            preferred_element_type=jnp.float32)
        o_ref[...] = y * pr_ref[...]


def _grouped_gemm(x_sorted, W1, W2, prob_sorted, tile_expert, tile_valid):
    grid_spec = pltpu.PrefetchScalarGridSpec(
        num_scalar_prefetch=2,
        grid=(MAXT,),
        in_specs=[
            pl.BlockSpec((TM, D), lambda t, te, tv: (t, 0)),
            pl.BlockSpec((1, F, D), lambda t, te, tv: (te[t], 0, 0)),
            pl.BlockSpec((1, D, F), lambda t, te, tv: (te[t], 0, 0)),
            pl.BlockSpec((TM, 1), lambda t, te, tv: (t, 0)),
        ],
        out_specs=pl.BlockSpec((TM, D), lambda t, te, tv: (t, 0)),
    )
    return pl.pallas_call(
        _gemm_body,
        grid_spec=grid_spec,
        out_shape=jax.ShapeDtypeStruct((MAXR, D), jnp.float32),
    )(tile_expert, tile_valid, x_sorted, W1, W2,
      prob_sorted.reshape(MAXR, 1))


def kernel(x, Wr, W1, W2):
    Bb, Tt, Dm = x.shape
    x_flat = x.reshape(N, D)
    i0, i1, p0, p1, aux, x_copy = _router(x_flat, Wr)
    gather_idx, prob_sorted, tile_expert, tile_valid, pos0, pos1 = _dispatch(
        i0, i1, p0, p1)
    x_sorted = _sc_gather_rows(x_copy, gather_idx)
    rows = _grouped_gemm(x_sorted, W1, W2, prob_sorted, tile_expert,
                         tile_valid)
    out = _sc_combine(rows, pos0, pos1)
    return (out.reshape(Bb, Tt, Dm), aux)


# speedup vs baseline: 1.4353x; 1.4353x over previous
import functools

import jax
import jax.numpy as jnp
from jax import lax
from jax.experimental import pallas as pl
from jax.experimental.pallas import tpu as pltpu
from jax.experimental.pallas import tpu_sc as plsc

N = 2048
D = 768
E = 6
K = 2
F = 3072
TM = 256
MAXT = (N * K) // TM + (E - 1)
MAXR = MAXT * TM
TN = 256


def _router_body(x_ref, wr_ref, i0_ref, i1_ref, p0_ref, p1_ref, aux_ref,
                 xc_ref, acc_ref):
    step = pl.program_id(0)
    xc_ref[...] = x_ref[...]

    @pl.when(step == 0)
    def _():
        acc_ref[...] = jnp.zeros_like(acc_ref)

    x = x_ref[...]
    wr = wr_ref[...]
    logits = lax.dot_general(x, wr, (((1,), (1,)), ((), ())),
                             preferred_element_type=jnp.float32)
    m = jnp.max(logits, axis=1, keepdims=True)
    ex = jnp.exp(logits - m)
    s = jnp.sum(ex, axis=1, keepdims=True)
    probs = ex / s

    iota = lax.broadcasted_iota(jnp.int32, (TN, E), 1)
    m1 = jnp.max(probs, axis=1, keepdims=True)
    i1 = jnp.min(jnp.where(probs == m1, iota, E), axis=1, keepdims=True)
    probs2 = jnp.where(iota == i1, -1.0, probs)
    m2 = jnp.max(probs2, axis=1, keepdims=True)
    i2 = jnp.min(jnp.where(probs2 == m2, iota, E), axis=1, keepdims=True)
    denom = m1 + m2
    i0_ref[...] = i1
    i1_ref[...] = i2
    p0_ref[...] = m1 / denom
    p1_ref[...] = m2 / denom

    acc_ref[0:1, 0:E] += jnp.sum(probs, axis=0, keepdims=True)

    @pl.when(step == pl.num_programs(0) - 1)
    def _():
        colmean = acc_ref[0:1, 0:E] / float(N)
        d = colmean - (1.0 / E)
        aux_ref[...] = (0.01 * jnp.sum(d * d) / float(E)).reshape(1, 1)


def _router(x_flat, Wr):
    grid = N // TN
    out_shapes = (
        jax.ShapeDtypeStruct((N, 1), jnp.int32),
        jax.ShapeDtypeStruct((N, 1), jnp.int32),
        jax.ShapeDtypeStruct((N, 1), jnp.float32),
        jax.ShapeDtypeStruct((N, 1), jnp.float32),
        jax.ShapeDtypeStruct((1, 1), jnp.float32),
        jax.ShapeDtypeStruct((N, D), jnp.float32),
    )
    tok_spec = pl.BlockSpec((TN, 1), lambda i: (i, 0))
    i0, i1, p0, p1, aux, x_copy = pl.pallas_call(
        _router_body,
        grid=(grid,),
        in_specs=[
            pl.BlockSpec((TN, D), lambda i: (i, 0)),
            pl.BlockSpec((E, D), lambda i: (0, 0)),
        ],
        out_specs=(tok_spec, tok_spec, tok_spec, tok_spec,
                   pl.BlockSpec((1, 1), lambda i: (0, 0)),
                   pl.BlockSpec((TN, D), lambda i: (i, 0))),
        out_shape=out_shapes,
        scratch_shapes=[pltpu.VMEM((8, 128), jnp.float32)],
    )(x_flat, Wr)
    return (i0[:, 0], i1[:, 0], p0[:, 0], p1[:, 0], aux[0, 0], x_copy)


L = 16
NV = N // L
MAXTP = 32


def _dispatch(i0, i1, p0, p1):
    mesh = plsc.VectorSubcoreMesh(core_axis_name="c", subcore_axis_name="s")
    SEGCAP = N + TM
    ZCH = MAXR // L

    @functools.partial(
        pl.kernel,
        out_type=(
            jax.ShapeDtypeStruct((MAXR,), jnp.int32),
            jax.ShapeDtypeStruct((MAXR,), jnp.float32),
            jax.ShapeDtypeStruct((MAXTP,), jnp.int32),
            jax.ShapeDtypeStruct((MAXTP,), jnp.int32),
            jax.ShapeDtypeStruct((N,), jnp.int32),
            jax.ShapeDtypeStruct((N,), jnp.int32),
        ),
        mesh=mesh,
        scratch_types=[
            pltpu.VMEM((N,), jnp.int32),
            pltpu.VMEM((N,), jnp.int32),
            pltpu.VMEM((N,), jnp.float32),
            pltpu.VMEM((N,), jnp.float32),
            pltpu.VMEM((SEGCAP,), jnp.int32),
            pltpu.VMEM((SEGCAP,), jnp.float32),
            pltpu.VMEM((N,), jnp.int32),
            pltpu.VMEM((N,), jnp.int32),
            pltpu.VMEM((MAXTP,), jnp.int32),
            pltpu.VMEM((MAXTP,), jnp.int32),
            pltpu.VMEM((N,), jnp.int32),
            pltpu.VMEM_SHARED((N,), jnp.int32),
            pltpu.VMEM_SHARED((N,), jnp.int32),
        ],
        compiler_params=pltpu.CompilerParams(needs_layout_passes=False),
    )
    def disp(i0_h, i1_h, p0_h, p1_h, g_h, ps_h, te_h, tv_h, pos0_h, pos1_h,
             vi0, vi1, vp0, vp1, seg_tok, seg_prob, vpos0, vpos1, texp_v,
             tval_v, idx_iota, pos0_sh, pos1_sh):
        cid = lax.axis_index("c")
        sid = lax.axis_index("s")
        z16i = jnp.zeros((L,), jnp.int32)
        z16f = jnp.zeros((L,), jnp.float32)

        @pl.when(cid == 0)
        def _():
            @pl.when(sid < E)
            def _():
                def bz(j, c):
                    vpos0[pl.ds(j * L, L)] = z16i
                    vpos1[pl.ds(j * L, L)] = z16i
                    idx_iota[pl.ds(j * L, L)] = lax.iota(jnp.int32, L) + j * L
                    return c

                lax.fori_loop(0, NV, bz, 0)

            def bseg(j, c):
                seg_tok[pl.ds(j * L, L)] = z16i
                seg_prob[pl.ds(j * L, L)] = z16f
                return c

            lax.fori_loop(0, SEGCAP // L, bseg, 0)

            pltpu.sync_copy(seg_tok.at[pl.ds(0, ZCH)],
                            g_h.at[pl.ds(sid * ZCH, ZCH)])
            pltpu.sync_copy(seg_prob.at[pl.ds(0, ZCH)],
                            ps_h.at[pl.ds(sid * ZCH, ZCH)])

            @pl.when(sid == 0)
            def _():
                pltpu.sync_copy(vpos0, pos0_sh)
                pltpu.sync_copy(vpos1, pos1_sh)

            @pl.when(sid < E)
            def _():
                pltpu.sync_copy(i0_h, vi0)
                pltpu.sync_copy(i1_h, vi1)
                pltpu.sync_copy(p0_h, vp0)
                pltpu.sync_copy(p1_h, vp1)

            plsc.subcore_barrier()

            @pl.when(sid < E)
            def _():
                def body1(j, accs):
                    off = j * L
                    v0 = vi0[pl.ds(off, L)]
                    v1 = vi1[pl.ds(off, L)]
                    return tuple(
                        accs[e]
                        + jnp.where(v0 == e, 1, 0).astype(jnp.int32)
                        + jnp.where(v1 == e, 1, 0).astype(jnp.int32)
                        for e in range(E))

                accs = lax.fori_loop(0, NV, body1,
                                     tuple(z16i for _ in range(E)))
                cnts = [jnp.sum(accs[e]) for e in range(E)]

                p_run = jnp.int32(0)
                t_run = jnp.int32(0)
                ppad, toff, tiles_l = [], [], []
                for e in range(E):
                    t_e = (cnts[e] + (TM - 1)) // TM
                    ppad.append(p_run)
                    toff.append(t_run)
                    tiles_l.append(t_e)
                    p_run = p_run + t_e * TM
                    t_run = t_run + t_e

                my_base = jnp.int32(0)
                my_tiles = jnp.int32(0)
                for e in range(E):
                    my_base = jnp.where(sid == e, ppad[e], my_base)
                    my_tiles = jnp.where(sid == e, tiles_l[e], my_tiles)

                def proc(v, pvals, n_ids, lbase, vpos):
                    m = v == sid
                    ones = jnp.where(m, 1, 0).astype(jnp.int32)
                    pref = plsc.cumsum(ones)
                    lpos = lbase + pref - 1
                    plsc.store_scatter(seg_tok, [lpos], n_ids, mask=m)
                    plsc.store_scatter(seg_prob, [lpos], pvals, mask=m)
                    plsc.store_scatter(vpos, [n_ids], lpos + my_base, mask=m)
                    return lbase + plsc.all_reduce_population_count(m)

                def body2(j, lbase):
                    off = j * L
                    n_ids = lax.iota(jnp.int32, L) + off
                    lbase = proc(vi0[pl.ds(off, L)], vp0[pl.ds(off, L)],
                                 n_ids, lbase, vpos0)
                    lbase = proc(vi1[pl.ds(off, L)], vp1[pl.ds(off, L)],
                                 n_ids, lbase, vpos1)
                    return lbase

                lax.fori_loop(0, NV, body2, z16i)

                def bcopy(j, c):
                    pltpu.sync_copy(
                        seg_tok.at[pl.ds(j * TM, TM)],
                        g_h.at[pl.ds(my_base + j * TM, TM)])
                    pltpu.sync_copy(
                        seg_prob.at[pl.ds(j * TM, TM)],
                        ps_h.at[pl.ds(my_base + j * TM, TM)])
                    return c

                lax.fori_loop(0, my_tiles, bcopy, 0)

                pltpu.sync_copy(vpos0, pos0_sh.at[idx_iota], add=True)
                pltpu.sync_copy(vpos1, pos1_sh.at[idx_iota], add=True)

                @pl.when(sid == 0)
                def _():
                    for half in range(MAXTP // L):
                        t16 = lax.iota(jnp.int32, L) + half * L
                        texp = jnp.full((L,), -1, jnp.int32)
                        for e in range(E):
                            texp = texp + jnp.where(
                                t16 >= toff[e], 1, 0).astype(jnp.int32)
                        tval = jnp.where(t16 < t_run, 1, 0).astype(jnp.int32)
                        texp_v[pl.ds(half * L, L)] = texp
                        tval_v[pl.ds(half * L, L)] = tval
                    pltpu.sync_copy(texp_v, te_h)
                    pltpu.sync_copy(tval_v, tv_h)

            plsc.subcore_barrier()

            @pl.when(sid == 0)
            def _():
                pltpu.sync_copy(pos0_sh, pos0_h)
                pltpu.sync_copy(pos1_sh, pos1_h)

    g, ps, te, tv, pos0, pos1 = disp(i0, i1, p0, p1)
    return g, ps, te[:MAXT], tv[:MAXT], pos0, pos1


def _sc_scatter_rows(x_copy, pos0, pos1):
    mesh = plsc.VectorSubcoreMesh(core_axis_name="c", subcore_axis_name="s")
    tpw = N // 32

    @functools.partial(
        pl.kernel,
        out_type=jax.ShapeDtypeStruct((MAXR, D), jnp.float32),
        mesh=mesh,
        scratch_types=[
            pltpu.VMEM((tpw,), jnp.int32),
            pltpu.VMEM((tpw,), jnp.int32),
            pltpu.VMEM((tpw, D), jnp.float32),
            pltpu.SemaphoreType.DMA,
            pltpu.SemaphoreType.DMA,
        ],
    )
    def scat(x_h, pos0_h, pos1_h, out_h, idx0, idx1, a_v, s0, s1):
        wid = lax.axis_index("s") * 2 + lax.axis_index("c")
        base = wid * tpw
        pltpu.sync_copy(pos0_h.at[pl.ds(base, tpw)], idx0)
        pltpu.sync_copy(pos1_h.at[pl.ds(base, tpw)], idx1)
        pltpu.sync_copy(x_h.at[pl.ds(base, tpw)], a_v)
        cp0 = pltpu.async_copy(a_v, out_h.at[idx0], s0)
        cp1 = pltpu.async_copy(a_v, out_h.at[idx1], s1)
        cp0.wait()
        cp1.wait()

    return scat(x_copy, pos0, pos1)


def _sc_gather_rows(x_flat, gather_idx):
    mesh = plsc.VectorSubcoreMesh(core_axis_name="c", subcore_axis_name="s")
    rpw = MAXR // 32
    ch = 56
    nch = rpw // ch

    @functools.partial(
        pl.kernel,
        out_type=jax.ShapeDtypeStruct((MAXR, D), jnp.float32),
        mesh=mesh,
        scratch_types=[
            pltpu.VMEM((ch,), jnp.int32),
            pltpu.VMEM((ch,), jnp.int32),
            pltpu.VMEM((ch,), jnp.int32),
            pltpu.VMEM((ch, D), jnp.float32),
            pltpu.VMEM((ch, D), jnp.float32),
            pltpu.SemaphoreType.DMA,
            pltpu.SemaphoreType.DMA,
            pltpu.SemaphoreType.DMA,
            pltpu.SemaphoreType.DMA,
        ],
    )
    def gat(x_h, gi_h, out_h, idx0, idx1, idx2, rows0, rows1, g0, g1, w0,
            w1):
        wid = lax.axis_index("s") * 2 + lax.axis_index("c")
        base = wid * rpw
        idxs = (idx0, idx1, idx2)
        for c in range(nch):
            pltpu.sync_copy(gi_h.at[pl.ds(base + c * ch, ch)], idxs[c])
        bufs = (rows0, rows1)
        gsems = (g0, g1)
        wsems = (w0, w1)
        gathers = [None] * nch
        writes = [None] * nch
        for c in range(nch):
            if c >= 2 and writes[c - 2] is not None:
                writes[c - 2].wait()
            gathers[c] = pltpu.async_copy(
                x_h.at[idxs[c]], bufs[c % 2], gsems[c % 2])
            if c >= 1:
                gathers[c - 1].wait()
                writes[c - 1] = pltpu.async_copy(
                    bufs[(c - 1) % 2],
                    out_h.at[pl.ds(base + (c - 1) * ch, ch)],
                    wsems[(c - 1) % 2])
        gathers[nch - 1].wait()
        writes[nch - 1] = pltpu.async_copy(
            bufs[(nch - 1) % 2],
            out_h.at[pl.ds(base + (nch - 1) * ch, ch)], wsems[(nch - 1) % 2])
        for c in (nch - 2, nch - 1):
            writes[c].wait()

    return gat(x_flat, gather_idx)


def _sc_combine(rows, pos0, pos1):
    mesh = plsc.VectorSubcoreMesh(core_axis_name="c", subcore_axis_name="s")
    tpw = N // 32

    @functools.partial(
        pl.kernel,
        out_type=jax.ShapeDtypeStruct((N, D), jnp.float32),
        mesh=mesh,
        scratch_types=[
            pltpu.VMEM((tpw,), jnp.int32),
            pltpu.VMEM((tpw,), jnp.int32),
            pltpu.VMEM((tpw, D), jnp.float32),
            pltpu.VMEM((tpw, D), jnp.float32),
            pltpu.SemaphoreType.DMA,
            pltpu.SemaphoreType.DMA,
        ],
    )
    def comb(rows_h, pos0_h, pos1_h, out_h, idx0_v, idx1_v, a_v, b_v,
             sem0, sem1):
        wid = lax.axis_index("s") * 2 + lax.axis_index("c")
        base = wid * tpw
        pltpu.sync_copy(pos0_h.at[pl.ds(base, tpw)], idx0_v)
        pltpu.sync_copy(pos1_h.at[pl.ds(base, tpw)], idx1_v)
        cp0 = pltpu.async_copy(rows_h.at[idx0_v], a_v, sem0)
        cp1 = pltpu.async_copy(rows_h.at[idx1_v], b_v, sem1)
        cp0.wait()
        cp1.wait()

        @plsc.parallel_loop(0, tpw, 1, unroll=2)
        def _add(r):
            for c in range(D // L):
                sl = pl.ds(c * L, L)
                a_v[r, sl] = a_v[r, sl] + b_v[r, sl]

        pltpu.sync_copy(a_v, out_h.at[pl.ds(base, tpw)])

    return comb(rows, pos0, pos1)


def _gemm_body(te_ref, tv_ref, x_ref, w1_ref, w2_ref, pr_ref, o_ref):
    t = pl.program_id(0)

    @pl.when(tv_ref[t] > 0)
    def _():
        x = x_ref[...]
        w1 = w1_ref[0]
        h = lax.dot_general(x, w1, (((1,), (1,)), ((), ())),
                            preferred_element_type=jnp.float32)
        g = 0.5 * h * (1.0 + lax.erf(h * 0.7071067811865476))
        w2 = w2_ref[0]
        y = lax.dot_general(g, w2, (((1,), (1,)), ((), ())),
                            preferred_element_type=jnp.float32)
        o_ref[...] = y * pr_ref[...]


def _grouped_gemm(x_sorted, W1, W2, prob_sorted, tile_expert, tile_valid):
    grid_spec = pltpu.PrefetchScalarGridSpec(
        num_scalar_prefetch=2,
        grid=(MAXT,),
        in_specs=[
            pl.BlockSpec((TM, D), lambda t, te, tv: (t, 0)),
            pl.BlockSpec((1, F, D), lambda t, te, tv: (te[t], 0, 0)),
            pl.BlockSpec((1, D, F), lambda t, te, tv: (te[t], 0, 0)),
            pl.BlockSpec((TM, 1), lambda t, te, tv: (t, 0)),
        ],
        out_specs=pl.BlockSpec((TM, D), lambda t, te, tv: (t, 0)),
    )
    return pl.pallas_call(
        _gemm_body,
        grid_spec=grid_spec,
        out_shape=jax.ShapeDtypeStruct((MAXR, D), jnp.float32),
    )(tile_expert, tile_valid, x_sorted, W1, W2,
      prob_sorted.reshape(MAXR, 1))


def kernel(x, Wr, W1, W2):
    Bb, Tt, Dm = x.shape
    x_flat = x.reshape(N, D)
    i0, i1, p0, p1, aux, x_copy = _router(x_flat, Wr)
    gather_idx, prob_sorted, tile_expert, tile_valid, pos0, pos1 = _dispatch(
        i0, i1, p0, p1)
    x_sorted = _sc_scatter_rows(x_copy, pos0, pos1)
    rows = _grouped_gemm(x_sorted, W1, W2, prob_sorted, tile_expert,
                         tile_valid)
    out = _sc_combine(rows, pos0, pos1)
    return (out.reshape(Bb, Tt, Dm), aux)


# speedup vs baseline: 1.4380x; 1.0019x over previous
import functools

import jax
import jax.numpy as jnp
from jax import lax
from jax.experimental import pallas as pl
from jax.experimental.pallas import tpu as pltpu
from jax.experimental.pallas import tpu_sc as plsc

N = 2048
D = 768
E = 6
K = 2
F = 3072
TM = 256
MAXT = (N * K) // TM + (E - 1)
MAXR = MAXT * TM
TN = 256


def _router_body(x_ref, wr_ref, i0_ref, i1_ref, p0_ref, p1_ref, aux_ref,
                 xc_ref, acc_ref):
    step = pl.program_id(0)
    xc_ref[...] = x_ref[...]

    @pl.when(step == 0)
    def _():
        acc_ref[...] = jnp.zeros_like(acc_ref)

    x = x_ref[...]
    wr = wr_ref[...]
    logits = lax.dot_general(x, wr, (((1,), (1,)), ((), ())),
                             preferred_element_type=jnp.float32)
    m = jnp.max(logits, axis=1, keepdims=True)
    ex = jnp.exp(logits - m)
    s = jnp.sum(ex, axis=1, keepdims=True)
    probs = ex / s

    iota = lax.broadcasted_iota(jnp.int32, (TN, E), 1)
    m1 = jnp.max(probs, axis=1, keepdims=True)
    i1 = jnp.min(jnp.where(probs == m1, iota, E), axis=1, keepdims=True)
    probs2 = jnp.where(iota == i1, -1.0, probs)
    m2 = jnp.max(probs2, axis=1, keepdims=True)
    i2 = jnp.min(jnp.where(probs2 == m2, iota, E), axis=1, keepdims=True)
    denom = m1 + m2
    i0_ref[...] = i1
    i1_ref[...] = i2
    p0_ref[...] = m1 / denom
    p1_ref[...] = m2 / denom

    acc_ref[0:1, 0:E] += jnp.sum(probs, axis=0, keepdims=True)

    @pl.when(step == pl.num_programs(0) - 1)
    def _():
        colmean = acc_ref[0:1, 0:E] / float(N)
        d = colmean - (1.0 / E)
        aux_ref[...] = (0.01 * jnp.sum(d * d) / float(E)).reshape(1, 1)


def _router(x_flat, Wr):
    grid = N // TN
    out_shapes = (
        jax.ShapeDtypeStruct((N, 1), jnp.int32),
        jax.ShapeDtypeStruct((N, 1), jnp.int32),
        jax.ShapeDtypeStruct((N, 1), jnp.float32),
        jax.ShapeDtypeStruct((N, 1), jnp.float32),
        jax.ShapeDtypeStruct((1, 1), jnp.float32),
        jax.ShapeDtypeStruct((N, D), jnp.float32),
    )
    tok_spec = pl.BlockSpec((TN, 1), lambda i: (i, 0))
    i0, i1, p0, p1, aux, x_copy = pl.pallas_call(
        _router_body,
        grid=(grid,),
        in_specs=[
            pl.BlockSpec((TN, D), lambda i: (i, 0)),
            pl.BlockSpec((E, D), lambda i: (0, 0)),
        ],
        out_specs=(tok_spec, tok_spec, tok_spec, tok_spec,
                   pl.BlockSpec((1, 1), lambda i: (0, 0)),
                   pl.BlockSpec((TN, D), lambda i: (i, 0))),
        out_shape=out_shapes,
        scratch_shapes=[pltpu.VMEM((8, 128), jnp.float32)],
    )(x_flat, Wr)
    return (i0.reshape(N), i1.reshape(N), p0.reshape(N), p1.reshape(N),
            aux.reshape(()), x_copy)


L = 16
NV = N // L
MAXTP = 32


def _dispatch(i0, i1, p0, p1):
    mesh = plsc.VectorSubcoreMesh(core_axis_name="c", subcore_axis_name="s")
    SEGCAP = N + TM
    ZCH = MAXR // L

    @functools.partial(
        pl.kernel,
        out_type=(
            jax.ShapeDtypeStruct((MAXR,), jnp.int32),
            jax.ShapeDtypeStruct((MAXR,), jnp.float32),
            jax.ShapeDtypeStruct((MAXTP,), jnp.int32),
            jax.ShapeDtypeStruct((MAXTP,), jnp.int32),
            jax.ShapeDtypeStruct((N,), jnp.int32),
            jax.ShapeDtypeStruct((N,), jnp.int32),
        ),
        mesh=mesh,
        scratch_types=[
            pltpu.VMEM((N,), jnp.int32),
            pltpu.VMEM((N,), jnp.int32),
            pltpu.VMEM((N,), jnp.float32),
            pltpu.VMEM((N,), jnp.float32),
            pltpu.VMEM((SEGCAP,), jnp.int32),
            pltpu.VMEM((SEGCAP,), jnp.float32),
            pltpu.VMEM((N,), jnp.int32),
            pltpu.VMEM((N,), jnp.int32),
            pltpu.VMEM((MAXTP,), jnp.int32),
            pltpu.VMEM((MAXTP,), jnp.int32),
            pltpu.VMEM((N,), jnp.int32),
            pltpu.VMEM_SHARED((N,), jnp.int32),
            pltpu.VMEM_SHARED((N,), jnp.int32),
        ],
        compiler_params=pltpu.CompilerParams(needs_layout_passes=False),
    )
    def disp(i0_h, i1_h, p0_h, p1_h, g_h, ps_h, te_h, tv_h, pos0_h, pos1_h,
             vi0, vi1, vp0, vp1, seg_tok, seg_prob, vpos0, vpos1, texp_v,
             tval_v, idx_iota, pos0_sh, pos1_sh):
        cid = lax.axis_index("c")
        sid = lax.axis_index("s")
        z16i = jnp.zeros((L,), jnp.int32)
        z16f = jnp.zeros((L,), jnp.float32)

        @pl.when(cid == 0)
        def _():
            @pl.when(sid < E)
            def _():
                def bz(j, c):
                    vpos0[pl.ds(j * L, L)] = z16i
                    vpos1[pl.ds(j * L, L)] = z16i
                    idx_iota[pl.ds(j * L, L)] = lax.iota(jnp.int32, L) + j * L
                    return c

                lax.fori_loop(0, NV, bz, 0)

            def bseg(j, c):
                seg_tok[pl.ds(j * L, L)] = z16i
                seg_prob[pl.ds(j * L, L)] = z16f
                return c

            lax.fori_loop(0, SEGCAP // L, bseg, 0)

            pltpu.sync_copy(seg_tok.at[pl.ds(0, ZCH)],
                            g_h.at[pl.ds(sid * ZCH, ZCH)])
            pltpu.sync_copy(seg_prob.at[pl.ds(0, ZCH)],
                            ps_h.at[pl.ds(sid * ZCH, ZCH)])

            @pl.when(sid == 0)
            def _():
                pltpu.sync_copy(vpos0, pos0_sh)
                pltpu.sync_copy(vpos1, pos1_sh)

            @pl.when(sid < E)
            def _():
                pltpu.sync_copy(i0_h, vi0)
                pltpu.sync_copy(i1_h, vi1)
                pltpu.sync_copy(p0_h, vp0)
                pltpu.sync_copy(p1_h, vp1)

            plsc.subcore_barrier()

            @pl.when(sid < E)
            def _():
                def body1(j, accs):
                    off = j * L
                    v0 = vi0[pl.ds(off, L)]
                    v1 = vi1[pl.ds(off, L)]
                    return tuple(
                        accs[e]
                        + jnp.where(v0 == e, 1, 0).astype(jnp.int32)
                        + jnp.where(v1 == e, 1, 0).astype(jnp.int32)
                        for e in range(E))

                accs = lax.fori_loop(0, NV, body1,
                                     tuple(z16i for _ in range(E)))
                cnts = [jnp.sum(accs[e]) for e in range(E)]

                p_run = jnp.int32(0)
                t_run = jnp.int32(0)
                ppad, toff, tiles_l = [], [], []
                for e in range(E):
                    t_e = (cnts[e] + (TM - 1)) // TM
                    ppad.append(p_run)
                    toff.append(t_run)
                    tiles_l.append(t_e)
                    p_run = p_run + t_e * TM
                    t_run = t_run + t_e

                my_base = jnp.int32(0)
                my_tiles = jnp.int32(0)
                for e in range(E):
                    my_base = jnp.where(sid == e, ppad[e], my_base)
                    my_tiles = jnp.where(sid == e, tiles_l[e], my_tiles)

                def proc(v, pvals, n_ids, lbase, vpos):
                    m = v == sid
                    ones = jnp.where(m, 1, 0).astype(jnp.int32)
                    pref = plsc.cumsum(ones)
                    lpos = lbase + pref - 1
                    plsc.store_scatter(seg_tok, [lpos], n_ids, mask=m)
                    plsc.store_scatter(seg_prob, [lpos], pvals, mask=m)
                    plsc.store_scatter(vpos, [n_ids], lpos + my_base, mask=m)
                    return lbase + plsc.all_reduce_population_count(m)

                def body2(j, lbase):
                    off = j * L
                    n_ids = lax.iota(jnp.int32, L) + off
                    lbase = proc(vi0[pl.ds(off, L)], vp0[pl.ds(off, L)],
                                 n_ids, lbase, vpos0)
                    lbase = proc(vi1[pl.ds(off, L)], vp1[pl.ds(off, L)],
                                 n_ids, lbase, vpos1)
                    return lbase

                lax.fori_loop(0, NV, body2, z16i)

                def bcopy(j, c):
                    pltpu.sync_copy(
                        seg_tok.at[pl.ds(j * TM, TM)],
                        g_h.at[pl.ds(my_base + j * TM, TM)])
                    pltpu.sync_copy(
                        seg_prob.at[pl.ds(j * TM, TM)],
                        ps_h.at[pl.ds(my_base + j * TM, TM)])
                    return c

                lax.fori_loop(0, my_tiles, bcopy, 0)

                pltpu.sync_copy(vpos0, pos0_sh.at[idx_iota], add=True)
                pltpu.sync_copy(vpos1, pos1_sh.at[idx_iota], add=True)

                @pl.when(sid == 0)
                def _():
                    for half in range(MAXTP // L):
                        t16 = lax.iota(jnp.int32, L) + half * L
                        texp = jnp.full((L,), -1, jnp.int32)
                        for e in range(E):
                            texp = texp + jnp.where(
                                t16 >= toff[e], 1, 0).astype(jnp.int32)
                        tval = jnp.where(t16 < t_run, 1, 0).astype(jnp.int32)
                        texp_v[pl.ds(half * L, L)] = texp
                        tval_v[pl.ds(half * L, L)] = tval
                    pltpu.sync_copy(texp_v, te_h)
                    pltpu.sync_copy(tval_v, tv_h)

            plsc.subcore_barrier()

            @pl.when(sid == 0)
            def _():
                pltpu.sync_copy(pos0_sh, pos0_h)
                pltpu.sync_copy(pos1_sh, pos1_h)

    g, ps, te, tv, pos0, pos1 = disp(i0, i1, p0, p1)
    return g, ps, te[:MAXT], tv[:MAXT], pos0, pos1


def _sc_scatter_rows(x_copy, pos0, pos1):
    mesh = plsc.VectorSubcoreMesh(core_axis_name="c", subcore_axis_name="s")
    tpw = N // 32

    @functools.partial(
        pl.kernel,
        out_type=jax.ShapeDtypeStruct((MAXR, D), jnp.float32),
        mesh=mesh,
        scratch_types=[
            pltpu.VMEM((tpw,), jnp.int32),
            pltpu.VMEM((tpw,), jnp.int32),
            pltpu.VMEM((tpw, D), jnp.float32),
            pltpu.SemaphoreType.DMA,
            pltpu.SemaphoreType.DMA,
        ],
    )
    def scat(x_h, pos0_h, pos1_h, out_h, idx0, idx1, a_v, s0, s1):
        wid = lax.axis_index("s") * 2 + lax.axis_index("c")
        base = wid * tpw
        pltpu.sync_copy(pos0_h.at[pl.ds(base, tpw)], idx0)
        pltpu.sync_copy(pos1_h.at[pl.ds(base, tpw)], idx1)
        pltpu.sync_copy(x_h.at[pl.ds(base, tpw)], a_v)
        cp0 = pltpu.async_copy(a_v, out_h.at[idx0], s0)
        cp1 = pltpu.async_copy(a_v, out_h.at[idx1], s1)
        cp0.wait()
        cp1.wait()

    return scat(x_copy, pos0, pos1)


def _sc_gather_rows(x_flat, gather_idx):
    mesh = plsc.VectorSubcoreMesh(core_axis_name="c", subcore_axis_name="s")
    rpw = MAXR // 32
    ch = 56
    nch = rpw // ch

    @functools.partial(
        pl.kernel,
        out_type=jax.ShapeDtypeStruct((MAXR, D), jnp.float32),
        mesh=mesh,
        scratch_types=[
            pltpu.VMEM((ch,), jnp.int32),
            pltpu.VMEM((ch,), jnp.int32),
            pltpu.VMEM((ch,), jnp.int32),
            pltpu.VMEM((ch, D), jnp.float32),
            pltpu.VMEM((ch, D), jnp.float32),
            pltpu.SemaphoreType.DMA,
            pltpu.SemaphoreType.DMA,
            pltpu.SemaphoreType.DMA,
            pltpu.SemaphoreType.DMA,
        ],
    )
    def gat(x_h, gi_h, out_h, idx0, idx1, idx2, rows0, rows1, g0, g1, w0,
            w1):
        wid = lax.axis_index("s") * 2 + lax.axis_index("c")
        base = wid * rpw
        idxs = (idx0, idx1, idx2)
        for c in range(nch):
            pltpu.sync_copy(gi_h.at[pl.ds(base + c * ch, ch)], idxs[c])
        bufs = (rows0, rows1)
        gsems = (g0, g1)
        wsems = (w0, w1)
        gathers = [None] * nch
        writes = [None] * nch
        for c in range(nch):
            if c >= 2 and writes[c - 2] is not None:
                writes[c - 2].wait()
            gathers[c] = pltpu.async_copy(
                x_h.at[idxs[c]], bufs[c % 2], gsems[c % 2])
            if c >= 1:
                gathers[c - 1].wait()
                writes[c - 1] = pltpu.async_copy(
                    bufs[(c - 1) % 2],
                    out_h.at[pl.ds(base + (c - 1) * ch, ch)],
                    wsems[(c - 1) % 2])
        gathers[nch - 1].wait()
        writes[nch - 1] = pltpu.async_copy(
            bufs[(nch - 1) % 2],
            out_h.at[pl.ds(base + (nch - 1) * ch, ch)], wsems[(nch - 1) % 2])
        for c in (nch - 2, nch - 1):
            writes[c].wait()

    return gat(x_flat, gather_idx)


def _sc_combine(rows, pos0, pos1):
    mesh = plsc.VectorSubcoreMesh(core_axis_name="c", subcore_axis_name="s")
    tpw = N // 32

    @functools.partial(
        pl.kernel,
        out_type=jax.ShapeDtypeStruct((N, D), jnp.float32),
        mesh=mesh,
        scratch_types=[
            pltpu.VMEM((tpw,), jnp.int32),
            pltpu.VMEM((tpw,), jnp.int32),
            pltpu.VMEM((tpw, D), jnp.float32),
            pltpu.VMEM((tpw, D), jnp.float32),
            pltpu.SemaphoreType.DMA,
            pltpu.SemaphoreType.DMA,
        ],
    )
    def comb(rows_h, pos0_h, pos1_h, out_h, idx0_v, idx1_v, a_v, b_v,
             sem0, sem1):
        wid = lax.axis_index("s") * 2 + lax.axis_index("c")
        base = wid * tpw
        pltpu.sync_copy(pos0_h.at[pl.ds(base, tpw)], idx0_v)
        pltpu.sync_copy(pos1_h.at[pl.ds(base, tpw)], idx1_v)
        cp0 = pltpu.async_copy(rows_h.at[idx0_v], a_v, sem0)
        cp1 = pltpu.async_copy(rows_h.at[idx1_v], b_v, sem1)
        cp0.wait()
        cp1.wait()

        @plsc.parallel_loop(0, tpw, 1, unroll=2)
        def _add(r):
            for c in range(D // L):
                sl = pl.ds(c * L, L)
                a_v[r, sl] = a_v[r, sl] + b_v[r, sl]

        pltpu.sync_copy(a_v, out_h.at[pl.ds(base, tpw)])

    return comb(rows, pos0, pos1)


def _gemm_body(te_ref, tv_ref, x_ref, w1_ref, w2_ref, pr_ref, o_ref):
    t = pl.program_id(0)

    @pl.when(tv_ref[t] > 0)
    def _():
        x = x_ref[...]
        w1 = w1_ref[0]
        h = lax.dot_general(x, w1, (((1,), (1,)), ((), ())),
                            preferred_element_type=jnp.float32)
        g = 0.5 * h * (1.0 + lax.erf(h * 0.7071067811865476))
        w2 = w2_ref[0]
        y = lax.dot_general(g, w2, (((1,), (1,)), ((), ())),
                            preferred_element_type=jnp.float32)
        o_ref[...] = y * pr_ref[...]


def _grouped_gemm(x_sorted, W1, W2, prob_sorted, tile_expert, tile_valid):
    grid_spec = pltpu.PrefetchScalarGridSpec(
        num_scalar_prefetch=2,
        grid=(MAXT,),
        in_specs=[
            pl.BlockSpec((TM, D), lambda t, te, tv: (t, 0)),
            pl.BlockSpec((1, F, D), lambda t, te, tv: (te[t], 0, 0)),
            pl.BlockSpec((1, D, F), lambda t, te, tv: (te[t], 0, 0)),
            pl.BlockSpec((TM, 1), lambda t, te, tv: (t, 0)),
        ],
        out_specs=pl.BlockSpec((TM, D), lambda t, te, tv: (t, 0)),
    )
    return pl.pallas_call(
        _gemm_body,
        grid_spec=grid_spec,
        out_shape=jax.ShapeDtypeStruct((MAXR, D), jnp.float32),
    )(tile_expert, tile_valid, x_sorted, W1, W2,
      prob_sorted.reshape(MAXR, 1))


def kernel(x, Wr, W1, W2):
    Bb, Tt, Dm = x.shape
    x_flat = x.reshape(N, D)
    i0, i1, p0, p1, aux, x_copy = _router(x_flat, Wr)
    gather_idx, prob_sorted, tile_expert, tile_valid, pos0, pos1 = _dispatch(
        i0, i1, p0, p1)
    x_sorted = _sc_scatter_rows(x_copy, pos0, pos1)
    rows = _grouped_gemm(x_sorted, W1, W2, prob_sorted, tile_expert,
                         tile_valid)
    out = _sc_combine(rows, pos0, pos1)
    return (out.reshape(Bb, Tt, Dm), aux)
